# speedup vs baseline: 1.0061x; 1.0061x over previous
import jax
import jax.numpy as jnp
import numpy as np
from jax.experimental import pallas as pl
from jax.experimental.pallas import tpu as pltpu

_B, _T, _N = 2, 128, 10000
_D = 64
_K = 20
_OUT_V = 10000
_EPS_BN = 1e-5


def _lin_body(x_ref, xo_ref, gt_ref):
    x = x_ref[...]
    mu = jnp.mean(x, axis=1, keepdims=True)
    var = jnp.mean((x - mu) ** 2, axis=1, keepdims=True)
    xn = (x - mu) / jnp.sqrt(var)
    bad = jnp.isinf(xn) | jnp.isnan(xn)
    xn = jnp.where(bad, 0.0, xn)
    gt_ref[...] = xn[:, -1:, :]
    t_iota = jax.lax.broadcasted_iota(jnp.int32, xn.shape, 1)
    xo_ref[...] = jnp.where(t_iota == _T - 1, 0.0, xn)


def _lin(x_enc):
    bn = 1024
    nblk = pl.cdiv(_N, bn)
    return pl.pallas_call(
        _lin_body,
        grid=(_B, nblk),
        in_specs=[pl.BlockSpec((1, _T, bn), lambda b, j: (b, 0, j))],
        out_specs=[
            pl.BlockSpec((1, _T, bn), lambda b, j: (b, 0, j)),
            pl.BlockSpec((1, 1, bn), lambda b, j: (b, 0, j)),
        ],
        out_shape=[
            jax.ShapeDtypeStruct((_B, _T, _N), jnp.float32),
            jax.ShapeDtypeStruct((_B, 1, _N), jnp.float32),
        ],
    )(x_enc)


def kernel(x_enc, emb_table, lin_W, att_i, att_j, att_em_i, att_em_j, gnn_bias,
           bn1_gamma, bn1_beta, bn2_gamma, bn2_beta, out_W, out_b):
    enc_input, gt = _lin(x_enc)
    enc = jnp.transpose(enc_input, (1, 0, 2))
    x = enc.reshape(-1, _T)

    weights = jax.lax.stop_gradient(emb_table)
    all_emb = jnp.tile(emb_table, (_B, 1))
    cos = weights @ weights.T
    nrm = jnp.sqrt(jnp.sum(weights * weights, axis=-1))
    cos = cos / (nrm[:, None] * nrm[None, :])
    _, topk_idx = jax.lax.top_k(cos, _K)
    gated_i = jnp.repeat(jnp.arange(_N), _K)
    gated_j = topk_idx.reshape(-1)
    offs = jnp.arange(_B) * _N
    src = (gated_j[None, :] + offs[:, None]).reshape(-1)
    dst = (gated_i[None, :] + offs[:, None]).reshape(-1)
    BN = _B * _N
    h = x @ lin_W
    self_idx = jnp.arange(BN)
    src_all = jnp.concatenate([src, self_idx])
    dst_all = jnp.concatenate([dst, self_idx])
    mask = jnp.concatenate([src != dst, jnp.ones((BN,), dtype=bool)])
    x_j = h[src_all]
    x_i = h[dst_all]
    emb_i = all_emb[dst_all]
    emb_j = all_emb[src_all]
    ai = att_i[0, 0]; aj = att_j[0, 0]; aei = att_em_i[0, 0]; aej = att_em_j[0, 0]
    alpha = x_i @ ai + emb_i @ aei + x_j @ aj + emb_j @ aej
    alpha = jnp.where(alpha > 0, alpha, 0.2 * alpha)
    am = jnp.where(mask, alpha, jnp.full_like(alpha, -1e30))
    smax = jax.ops.segment_max(am, dst_all, num_segments=BN)
    smax = jnp.where(smax < -1e29, 0.0, smax)
    ex = jnp.exp(am - smax[dst_all]) * mask.astype(alpha.dtype)
    den = jax.ops.segment_sum(ex, dst_all, num_segments=BN)
    w = ex / den[dst_all]
    agg = jax.ops.segment_sum(x_j * w[:, None], dst_all, num_segments=BN)
    out = agg + gnn_bias
    out = out / jnp.sqrt(1.0 + _EPS_BN) * bn1_gamma + bn1_beta
    out = jnp.maximum(out, 0.0)
    xg = out.reshape(_B, _N, _D)
    o = xg * emb_table[None, :, :]
    o = jnp.transpose(o, (0, 2, 1))
    o = o / jnp.sqrt(1.0 + _EPS_BN) * bn2_gamma[None, :, None] + bn2_beta[None, :, None]
    o = jnp.maximum(o, 0.0)
    o = jnp.transpose(o, (0, 2, 1))
    pred = o @ out_W.T + out_b
    pred = jnp.transpose(pred, (0, 2, 1))
    return pred[:, :, :_OUT_V], gt


# speedup vs baseline: 1.6488x; 1.6387x over previous
import jax
import jax.numpy as jnp
import numpy as np
from jax.experimental import pallas as pl
from jax.experimental.pallas import tpu as pltpu

_B, _T, _N = 2, 128, 10000
_D = 64
_K = 20
_OUT_V = 10000
_EPS_BN = 1e-5


def _lin_body(x_ref, xo_ref, gt_ref):
    x = x_ref[...]
    mu = jnp.mean(x, axis=1, keepdims=True)
    var = jnp.mean((x - mu) ** 2, axis=1, keepdims=True)
    xn = (x - mu) / jnp.sqrt(var)
    bad = jnp.isinf(xn) | jnp.isnan(xn)
    xn = jnp.where(bad, 0.0, xn)
    gt_ref[...] = xn[:, -1:, :]
    t_iota = jax.lax.broadcasted_iota(jnp.int32, xn.shape, 1)
    xo_ref[...] = jnp.where(t_iota == _T - 1, 0.0, xn)


def _lin(x_enc):
    bn = 1024
    nblk = pl.cdiv(_N, bn)
    return pl.pallas_call(
        _lin_body,
        grid=(_B, nblk),
        in_specs=[pl.BlockSpec((1, _T, bn), lambda b, j: (b, 0, j))],
        out_specs=[
            pl.BlockSpec((1, _T, bn), lambda b, j: (b, 0, j)),
            pl.BlockSpec((1, 1, bn), lambda b, j: (b, 0, j)),
        ],
        out_shape=[
            jax.ShapeDtypeStruct((_B, _T, _N), jnp.float32),
            jax.ShapeDtypeStruct((_B, 1, _N), jnp.float32),
        ],
    )(x_enc)


def _topk_body(emb_ref, embT_ref, out_ref, vals_ref):
    emb_blk = emb_ref[...]
    embT = embT_ref[...]
    gram = jnp.dot(emb_blk, embT, preferred_element_type=jnp.float32)
    nrm_blk = jnp.sqrt(jnp.sum(emb_blk * emb_blk, axis=1, keepdims=True))
    nrm_all = jnp.sqrt(jnp.sum(embT * embT, axis=0, keepdims=True))
    vals_ref[...] = gram / (nrm_blk * nrm_all)
    iota_j = jax.lax.broadcasted_iota(jnp.int32, vals_ref.shape, 1)
    cols = []
    for _ in range(_K):
        v = vals_ref[...]
        m = jnp.max(v, axis=1, keepdims=True)
        eq = v == m
        idxv = jnp.min(jnp.where(eq, iota_j, _N), axis=1, keepdims=True)
        cols.append(idxv)
        vals_ref[...] = jnp.where(eq & (iota_j == idxv), -2.0, v)
    out_ref[...] = jnp.concatenate(cols, axis=1)


def _topk(emb_table):
    R = 256
    embT = jnp.transpose(emb_table)
    return pl.pallas_call(
        _topk_body,
        grid=(pl.cdiv(_N, R),),
        in_specs=[
            pl.BlockSpec((R, _D), lambda i: (i, 0)),
            pl.BlockSpec((_D, _N), lambda i: (0, 0)),
        ],
        out_specs=pl.BlockSpec((R, _K), lambda i: (i, 0)),
        out_shape=jax.ShapeDtypeStruct((_N, _K), jnp.int32),
        scratch_shapes=[pltpu.VMEM((R, _N), jnp.float32)],
    )(emb_table, embT)


def kernel(x_enc, emb_table, lin_W, att_i, att_j, att_em_i, att_em_j, gnn_bias,
           bn1_gamma, bn1_beta, bn2_gamma, bn2_beta, out_W, out_b):
    enc_input, gt = _lin(x_enc)
    enc = jnp.transpose(enc_input, (1, 0, 2))
    x = enc.reshape(-1, _T)

    all_emb = jnp.tile(emb_table, (_B, 1))
    topk_idx = _topk(emb_table)
    gated_i = jnp.repeat(jnp.arange(_N), _K)
    gated_j = topk_idx.reshape(-1)
    offs = jnp.arange(_B) * _N
    src = (gated_j[None, :] + offs[:, None]).reshape(-1)
    dst = (gated_i[None, :] + offs[:, None]).reshape(-1)
    BN = _B * _N
    h = x @ lin_W
    self_idx = jnp.arange(BN)
    src_all = jnp.concatenate([src, self_idx])
    dst_all = jnp.concatenate([dst, self_idx])
    mask = jnp.concatenate([src != dst, jnp.ones((BN,), dtype=bool)])
    x_j = h[src_all]
    x_i = h[dst_all]
    emb_i = all_emb[dst_all]
    emb_j = all_emb[src_all]
    ai = att_i[0, 0]; aj = att_j[0, 0]; aei = att_em_i[0, 0]; aej = att_em_j[0, 0]
    alpha = x_i @ ai + emb_i @ aei + x_j @ aj + emb_j @ aej
    alpha = jnp.where(alpha > 0, alpha, 0.2 * alpha)
    am = jnp.where(mask, alpha, jnp.full_like(alpha, -1e30))
    smax = jax.ops.segment_max(am, dst_all, num_segments=BN)
    smax = jnp.where(smax < -1e29, 0.0, smax)
    ex = jnp.exp(am - smax[dst_all]) * mask.astype(alpha.dtype)
    den = jax.ops.segment_sum(ex, dst_all, num_segments=BN)
    w = ex / den[dst_all]
    agg = jax.ops.segment_sum(x_j * w[:, None], dst_all, num_segments=BN)
    out = agg + gnn_bias
    out = out / jnp.sqrt(1.0 + _EPS_BN) * bn1_gamma + bn1_beta
    out = jnp.maximum(out, 0.0)
    xg = out.reshape(_B, _N, _D)
    o = xg * emb_table[None, :, :]
    o = jnp.transpose(o, (0, 2, 1))
    o = o / jnp.sqrt(1.0 + _EPS_BN) * bn2_gamma[None, :, None] + bn2_beta[None, :, None]
    o = jnp.maximum(o, 0.0)
    o = jnp.transpose(o, (0, 2, 1))
    pred = o @ out_W.T + out_b
    pred = jnp.transpose(pred, (0, 2, 1))
    return pred[:, :, :_OUT_V], gt


# speedup vs baseline: 5.7780x; 3.5044x over previous
import functools

import jax
import jax.numpy as jnp
import numpy as np
from jax.experimental import pallas as pl
from jax.experimental.pallas import tpu as pltpu
from jax.experimental.pallas import tpu_sc as plsc

_B, _T, _N = 2, 128, 10000
_D = 64
_K = 20
_OUT_V = 10000
_EPS_BN = 1e-5


def _lin_body(x_ref, xo_ref, gt_ref):
    x = x_ref[...]
    mu = jnp.mean(x, axis=1, keepdims=True)
    var = jnp.mean((x - mu) ** 2, axis=1, keepdims=True)
    xn = (x - mu) / jnp.sqrt(var)
    bad = jnp.isinf(xn) | jnp.isnan(xn)
    xn = jnp.where(bad, 0.0, xn)
    gt_ref[...] = xn[:, -1:, :]
    t_iota = jax.lax.broadcasted_iota(jnp.int32, xn.shape, 1)
    xo_ref[...] = jnp.where(t_iota == _T - 1, 0.0, xn)


def _lin(x_enc):
    bn = 1024
    nblk = pl.cdiv(_N, bn)
    return pl.pallas_call(
        _lin_body,
        grid=(_B, nblk),
        in_specs=[pl.BlockSpec((1, _T, bn), lambda b, j: (b, 0, j))],
        out_specs=[
            pl.BlockSpec((1, _T, bn), lambda b, j: (b, 0, j)),
            pl.BlockSpec((1, 1, bn), lambda b, j: (b, 0, j)),
        ],
        out_shape=[
            jax.ShapeDtypeStruct((_B, _T, _N), jnp.float32),
            jax.ShapeDtypeStruct((_B, 1, _N), jnp.float32),
        ],
    )(x_enc)


def _topk_body(emb_ref, embT_ref, out_ref, vals_ref):
    emb_blk = emb_ref[...]
    embT = embT_ref[...]
    gram = jnp.dot(emb_blk, embT, preferred_element_type=jnp.float32)
    nrm_blk = jnp.sqrt(jnp.sum(emb_blk * emb_blk, axis=1, keepdims=True))
    nrm_all = jnp.sqrt(jnp.sum(embT * embT, axis=0, keepdims=True))
    vals_ref[...] = gram / (nrm_blk * nrm_all)
    iota_j = jax.lax.broadcasted_iota(jnp.int32, vals_ref.shape, 1)
    cols = []
    for _ in range(_K):
        v = vals_ref[...]
        m = jnp.max(v, axis=1, keepdims=True)
        eq = v == m
        idxv = jnp.min(jnp.where(eq, iota_j, _N), axis=1, keepdims=True)
        cols.append(idxv)
        vals_ref[...] = jnp.where(eq & (iota_j == idxv), -2.0, v)
    out_ref[...] = jnp.concatenate(cols, axis=1)


def _topk(emb_table):
    R = 256
    embT = jnp.transpose(emb_table)
    return pl.pallas_call(
        _topk_body,
        grid=(pl.cdiv(_N, R),),
        in_specs=[
            pl.BlockSpec((R, _D), lambda i: (i, 0)),
            pl.BlockSpec((_D, _N), lambda i: (0, 0)),
        ],
        out_specs=pl.BlockSpec((R, _K), lambda i: (i, 0)),
        out_shape=jax.ShapeDtypeStruct((_N, _K), jnp.int32),
        scratch_shapes=[pltpu.VMEM((R, _N), jnp.float32)],
    )(emb_table, embT)


_NW = 32
_P = 20480
_RPT = _P // _NW
_GS = 16
_GPT = _RPT // _GS
_E = _K + 1


def _agg_body(h_hbm, idx_hbm, qj_hbm, pi_hbm, out_hbm,
              idx_v, qj_v, pi_v, rows_v, aw_v, out_v, sem):
    c = jax.lax.axis_index("c")
    s = jax.lax.axis_index("s")
    wid = s * 2 + c
    base_row = wid * _RPT
    pltpu.sync_copy(idx_hbm.at[wid], idx_v)
    pltpu.sync_copy(qj_hbm, qj_v)
    pltpu.sync_copy(pi_hbm.at[pl.ds(base_row, _RPT)], pi_v)

    def group(g, carry):
        off = g * (_E * _GS)
        pltpu.async_copy(h_hbm.at[idx_v.at[pl.ds(off, _E * _GS)]], rows_v, sem).wait()
        lane = jax.lax.broadcasted_iota(jnp.int32, (_GS,), 0)
        r_vec = base_row + g * _GS + lane
        pi_vec = pi_v[pl.ds(g * _GS, _GS)]
        m = jnp.full((_GS,), -1e30, jnp.float32)
        for k in range(_E):
            idxk = idx_v[pl.ds(off + k * _GS, _GS)]
            qk = plsc.load_gather(qj_v, [idxk])
            a = pi_vec + qk
            a = jnp.where(a > 0, a, 0.2 * a)
            if k < _E - 1:
                a = jnp.where(idxk != r_vec, a, -1e30)
            aw_v[k] = a
            m = jnp.maximum(m, a)
        den = jnp.zeros((_GS,), jnp.float32)
        for k in range(_E):
            e = jnp.exp(aw_v[k] - m)
            aw_v[k] = e
            den = den + e
        for k in range(_E):
            aw_v[k] = aw_v[k] / den
        for n in range(_GS):
            accs = [jnp.zeros((16,), jnp.float32) for _ in range(4)]
            nsp = jnp.full((16,), n, jnp.int32)
            for k in range(_E):
                wbc = plsc.load_gather(aw_v, [jnp.full((16,), k, jnp.int32), nsp])
                for d in range(4):
                    accs[d] = accs[d] + wbc * rows_v[k * _GS + n, pl.ds(d * 16, 16)]
            for d in range(4):
                out_v[g * _GS + n, pl.ds(d * 16, 16)] = accs[d]
        return carry
    jax.lax.fori_loop(0, _GPT, group, 0)
    pltpu.sync_copy(out_v, out_hbm.at[pl.ds(base_row, _RPT)])


@jax.jit
def _sc_agg(h, idx_packed, qj_pad, pi_pad):
    mesh = plsc.VectorSubcoreMesh(core_axis_name="c", subcore_axis_name="s")
    f = pl.kernel(
        _agg_body,
        out_type=jax.ShapeDtypeStruct((_P, _D), jnp.float32),
        mesh=mesh,
        compiler_params=pltpu.CompilerParams(
            needs_layout_passes=False, use_tc_tiling_on_sc=False
        ),
        scratch_types=[
            pltpu.VMEM((_GPT * _E * _GS,), jnp.int32),
            pltpu.VMEM((_P,), jnp.float32),
            pltpu.VMEM((_RPT,), jnp.float32),
            pltpu.VMEM((_E * _GS, _D), jnp.float32),
            pltpu.VMEM((_E, _GS), jnp.float32),
            pltpu.VMEM((_RPT, _D), jnp.float32),
            pltpu.SemaphoreType.DMA,
        ],
    )
    return f(h, idx_packed, qj_pad, pi_pad)


def kernel(x_enc, emb_table, lin_W, att_i, att_j, att_em_i, att_em_j, gnn_bias,
           bn1_gamma, bn1_beta, bn2_gamma, bn2_beta, out_W, out_b):
    enc_input, gt = _lin(x_enc)
    enc = jnp.transpose(enc_input, (1, 0, 2))
    x = enc.reshape(-1, _T)

    topk_idx = _topk(emb_table)
    BN = _B * _N
    h = x @ lin_W
    ai = att_i[0, 0]; aj = att_j[0, 0]; aei = att_em_i[0, 0]; aej = att_em_j[0, 0]
    pe_i = emb_table @ aei
    pe_j = emb_table @ aej
    pi = h @ ai + jnp.tile(pe_i, (_B,))
    qj = h @ aj + jnp.tile(pe_j, (_B,))
    offs = (jnp.arange(_B) * _N)[:, None, None]
    nb = (topk_idx[None, :, :] + offs).reshape(BN, _K)
    idx21 = jnp.concatenate([nb, jnp.arange(BN)[:, None]], axis=1)
    idx21 = jnp.pad(idx21, ((0, _P - BN), (0, 0)))
    idx_packed = jnp.transpose(
        idx21.reshape(_NW, _GPT, _GS, _E), (0, 1, 3, 2)
    ).reshape(_NW, _GPT * _E * _GS)
    qj_pad = jnp.pad(qj, (0, _P - BN))
    pi_pad = jnp.pad(pi, (0, _P - BN))
    agg = _sc_agg(h, idx_packed, qj_pad, pi_pad)[:BN]
    out = agg + gnn_bias
    out = out / jnp.sqrt(1.0 + _EPS_BN) * bn1_gamma + bn1_beta
    out = jnp.maximum(out, 0.0)
    xg = out.reshape(_B, _N, _D)
    o = xg * emb_table[None, :, :]
    o = jnp.transpose(o, (0, 2, 1))
    o = o / jnp.sqrt(1.0 + _EPS_BN) * bn2_gamma[None, :, None] + bn2_beta[None, :, None]
    o = jnp.maximum(o, 0.0)
    o = jnp.transpose(o, (0, 2, 1))
    pred = o @ out_W.T + out_b
    pred = jnp.transpose(pred, (0, 2, 1))
    return pred[:, :, :_OUT_V], gt
